# baseline (speedup 1.0000x reference)
import functools

import numpy as np
import jax
import jax.numpy as jnp
from jax import lax
from jax.experimental import pallas as pl
from jax.experimental.pallas import tpu as pltpu
from jax.experimental.pallas import tpu_sc as plsc

_NUM_FIELDS = 26
_EMBED_DIM = 32
_NUM_NUM = 13
_BATCH = 16384
_FIELD_SIZE = 100000
_OFFSETS = np.arange(_NUM_FIELDS, dtype=np.int32) * _FIELD_SIZE

_NW = 32
_ROWS_PER_W = _BATCH // _NW
_CHUNK = 64
_NCHUNK = _ROWS_PER_W // _CHUNK

_ROWS = 2600000
_ROWS_PAD = 2600064
_SLAB = 512
_NSLAB = (_ROWS_PAD - 128) // _SLAB
_SLAB_REM = 22
_PACK = 128 // _EMBED_DIM
_OUT_ROWS = _ROWS_PAD // _PACK


_TBLK = 32768
_NTBLK = 80
_TSUB = _TBLK // 4


def _tc_transpose(tt_ref, out_ref):
    x = tt_ref[...]
    xs = jnp.concatenate(
        [x[:, q * _TSUB : (q + 1) * _TSUB] for q in range(4)], axis=0
    )
    out_ref[...] = jnp.transpose(xs)


def _tc_prep(cat_ref, x_ref, wt_ref, b_ref, off_ref, idx_ref, num_ref):
    r = cat_ref[...] + off_ref[...]
    idx_ref[...] = (
        (r & -_TBLK) + ((r & (_TSUB - 1)) << 2) + ((r & (_TBLK - 1)) >> 13)
    )
    num_ref[...] = (
        jnp.dot(x_ref[...], wt_ref[...], preferred_element_type=jnp.float32)
        + b_ref[...]
    )


_sc_mesh = plsc.VectorSubcoreMesh(core_axis_name="c", subcore_axis_name="s")


@functools.partial(
    pl.kernel,
    mesh=_sc_mesh,
    out_type=jax.ShapeDtypeStruct((_OUT_ROWS, 128), jnp.float32),
    scratch_types=[
        pltpu.VMEM((2 * 32, _SLAB), jnp.float32),
        pltpu.VMEM((2 * (_SLAB // _PACK), 128), jnp.float32),
        pltpu.SemaphoreType.DMA,
        pltpu.SemaphoreType.DMA,
    ],
    compiler_params=pltpu.CompilerParams(
        use_tc_tiling_on_sc=True,
        needs_layout_passes=False,
        disable_bounds_checks=True,
    ),
)
def _sc_relayout(tt_hbm, tail_hbm, out_hbm, in2, out2, in_sem, out_sem):
    wid = lax.axis_index("s") * 2 + lax.axis_index("c")
    n_w = jnp.where(wid < _SLAB_REM, _NSLAB // _NW + 1, _NSLAB // _NW)
    iota = lax.iota(jnp.int32, 16)
    _UJ = 8

    def transpose_slab(p):
        def tj(jb, carry):
            for jj in range(_UJ):
                j = jb * _UJ + jj
                for k in range(8):
                    idx_d = iota + (p * 32 + 16 * (k % 2))
                    idx_r = jnp.full((16,), 0, jnp.int32) + (_PACK * j + k // 2)
                    vec = plsc.load_gather(in2, [idx_d, idx_r])
                    out2[p * (_SLAB // _PACK) + j, pl.ds(k * 16, 16)] = vec
            return carry

        lax.fori_loop(0, (_SLAB // _PACK) // _UJ, tj, 0)

    pltpu.async_copy(
        tt_hbm.at[pl.ds(0, 32), pl.ds(wid * _SLAB, _SLAB)],
        in2.at[pl.ds(0, 32)],
        in_sem,
    )

    def body(i, carry):
        p = lax.rem(i, 2)
        s = wid + _NW * i
        pltpu.make_async_copy(
            tt_hbm.at[pl.ds(0, 32), pl.ds(s * _SLAB, _SLAB)],
            in2.at[pl.ds(p * 32, 32)],
            in_sem,
        ).wait()

        @pl.when(i + 1 < n_w)
        def _():
            pltpu.async_copy(
                tt_hbm.at[pl.ds(0, 32), pl.ds((s + _NW) * _SLAB, _SLAB)],
                in2.at[pl.ds((1 - p) * 32, 32)],
                in_sem,
            )

        @pl.when(i >= 2)
        def _():
            pltpu.make_async_copy(
                tt_hbm.at[pl.ds(0, 32), pl.ds(0, _SLAB)],
                in2.at[pl.ds(p * 32, 32)],
                out_sem,
            ).wait()

        transpose_slab(p)
        pltpu.async_copy(
            out2.at[pl.ds(p * (_SLAB // _PACK), _SLAB // _PACK)],
            out_hbm.at[pl.ds(s * (_SLAB // _PACK), _SLAB // _PACK)],
            out_sem,
        )
        return carry

    lax.fori_loop(0, n_w, body, 0)

    for _ in range(2):
        pltpu.make_async_copy(
            tt_hbm.at[pl.ds(0, 32), pl.ds(0, _SLAB)],
            in2.at[pl.ds(0, 32)],
            out_sem,
        ).wait()

    @pl.when(wid == _NW - 1)
    def _():
        pltpu.sync_copy(tail_hbm, out2.at[pl.ds(0, 16)])
        pltpu.sync_copy(
            out2.at[pl.ds(0, 16)],
            out_hbm.at[pl.ds(_NSLAB * _SLAB // _PACK, 16)],
        )


@functools.partial(
    pl.kernel,
    mesh=_sc_mesh,
    out_type=jax.ShapeDtypeStruct((_BATCH, _NUM_FIELDS + 2, _EMBED_DIM), jnp.float32),
    scratch_types=[
        pltpu.VMEM((_CHUNK, _NUM_FIELDS), jnp.int32),
        pltpu.VMEM((_CHUNK, _NUM_FIELDS + 2, _EMBED_DIM), jnp.float32),
        pltpu.SemaphoreType.DMA,
    ],
    compiler_params=pltpu.CompilerParams(use_tc_tiling_on_sc=False),
)
def _sc_gather(idx_hbm, num_hbm, table_hbm, out_hbm, idx_v, full_v, sem):
    wid = lax.axis_index("s") * 2 + lax.axis_index("c")

    def chunk_body(c, carry):
        base = wid * _ROWS_PER_W + c * _CHUNK
        pltpu.sync_copy(idx_hbm.at[pl.ds(base, _CHUNK)], idx_v)
        cps = [
            pltpu.async_copy(
                table_hbm.at[idx_v.at[r]],
                full_v.at[r, pl.ds(0, _NUM_FIELDS)],
                sem,
            )
            for r in range(_CHUNK)
        ]
        pltpu.sync_copy(
            num_hbm.at[pl.ds(base, _CHUNK)],
            full_v.at[pl.ds(0, _CHUNK), _NUM_FIELDS],
        )
        for cp in cps:
            cp.wait()
        pltpu.sync_copy(full_v, out_hbm.at[pl.ds(base, _CHUNK)])
        return carry

    lax.fori_loop(0, _NCHUNK, chunk_body, 0)


def kernel(categorical_x, numerical_x, table, W, b):
    offsets = jnp.asarray(_OFFSETS)[None, :]
    wt = W.T
    b2 = b[None, :]
    idx, num = pl.pallas_call(
        _tc_prep,
        out_shape=(
            jax.ShapeDtypeStruct((_BATCH, _NUM_FIELDS), jnp.int32),
            jax.ShapeDtypeStruct((_BATCH, _EMBED_DIM), jnp.float32),
        ),
    )(categorical_x, numerical_x, wt, b2, offsets)
    packed = pl.pallas_call(
        _tc_transpose,
        grid=(_NTBLK,),
        in_specs=[pl.BlockSpec((32, _TBLK), lambda i: (0, i))],
        out_specs=pl.BlockSpec((_TBLK // _PACK, 128), lambda i: (i, 0)),
        out_shape=jax.ShapeDtypeStruct((_NTBLK * _TBLK // _PACK, 128), jnp.float32),
    )(table.T)
    table_lin = packed.reshape(_NTBLK * _TBLK, _EMBED_DIM)
    out = _sc_gather(idx, num, table_lin)
    wide = out.reshape(_BATCH, (_NUM_FIELDS + 2) * _EMBED_DIM)
    return wide[:, : (_NUM_FIELDS + 1) * _EMBED_DIM]


# speedup vs baseline: 1.0864x; 1.0864x over previous
import functools

import numpy as np
import jax
import jax.numpy as jnp
from jax import lax
from jax.experimental import pallas as pl
from jax.experimental.pallas import tpu as pltpu
from jax.experimental.pallas import tpu_sc as plsc

_NUM_FIELDS = 26
_EMBED_DIM = 32
_NUM_NUM = 13
_BATCH = 16384
_FIELD_SIZE = 100000
_OFFSETS = np.arange(_NUM_FIELDS, dtype=np.int32) * _FIELD_SIZE

_NW = 32
_ROWS_PER_W = _BATCH // _NW
_CHUNK = 64
_NCHUNK = _ROWS_PER_W // _CHUNK

_ROWS = 2600000
_ROWS_PAD = 2600064
_SLAB = 512
_NSLAB = (_ROWS_PAD - 128) // _SLAB
_SLAB_REM = 22
_PACK = 128 // _EMBED_DIM
_OUT_ROWS = _ROWS_PAD // _PACK


_TBLK = 32768
_NTBLK = 80
_TSUB = _TBLK // 4


def _tc_transpose(tt_ref, out_ref):
    x = tt_ref[...]
    xs = jnp.concatenate(
        [x[:, q * _TSUB : (q + 1) * _TSUB] for q in range(4)], axis=0
    )
    out_ref[...] = jnp.transpose(xs)


def _tc_prep(cat_ref, x_ref, wt_ref, b_ref, off_ref, idx_ref, num_ref):
    r = cat_ref[...] + off_ref[...]
    idx_ref[...] = (
        (r & -_TBLK) + ((r & (_TSUB - 1)) << 2) + ((r & (_TBLK - 1)) >> 13)
    )
    num_ref[...] = (
        jnp.dot(x_ref[...], wt_ref[...], preferred_element_type=jnp.float32)
        + b_ref[...]
    )


_sc_mesh = plsc.VectorSubcoreMesh(core_axis_name="c", subcore_axis_name="s")


@functools.partial(
    pl.kernel,
    mesh=_sc_mesh,
    out_type=jax.ShapeDtypeStruct((_OUT_ROWS, 128), jnp.float32),
    scratch_types=[
        pltpu.VMEM((2 * 32, _SLAB), jnp.float32),
        pltpu.VMEM((2 * (_SLAB // _PACK), 128), jnp.float32),
        pltpu.SemaphoreType.DMA,
        pltpu.SemaphoreType.DMA,
    ],
    compiler_params=pltpu.CompilerParams(
        use_tc_tiling_on_sc=True,
        needs_layout_passes=False,
        disable_bounds_checks=True,
    ),
)
def _sc_relayout(tt_hbm, tail_hbm, out_hbm, in2, out2, in_sem, out_sem):
    wid = lax.axis_index("s") * 2 + lax.axis_index("c")
    n_w = jnp.where(wid < _SLAB_REM, _NSLAB // _NW + 1, _NSLAB // _NW)
    iota = lax.iota(jnp.int32, 16)
    _UJ = 8

    def transpose_slab(p):
        def tj(jb, carry):
            for jj in range(_UJ):
                j = jb * _UJ + jj
                for k in range(8):
                    idx_d = iota + (p * 32 + 16 * (k % 2))
                    idx_r = jnp.full((16,), 0, jnp.int32) + (_PACK * j + k // 2)
                    vec = plsc.load_gather(in2, [idx_d, idx_r])
                    out2[p * (_SLAB // _PACK) + j, pl.ds(k * 16, 16)] = vec
            return carry

        lax.fori_loop(0, (_SLAB // _PACK) // _UJ, tj, 0)

    pltpu.async_copy(
        tt_hbm.at[pl.ds(0, 32), pl.ds(wid * _SLAB, _SLAB)],
        in2.at[pl.ds(0, 32)],
        in_sem,
    )

    def body(i, carry):
        p = lax.rem(i, 2)
        s = wid + _NW * i
        pltpu.make_async_copy(
            tt_hbm.at[pl.ds(0, 32), pl.ds(s * _SLAB, _SLAB)],
            in2.at[pl.ds(p * 32, 32)],
            in_sem,
        ).wait()

        @pl.when(i + 1 < n_w)
        def _():
            pltpu.async_copy(
                tt_hbm.at[pl.ds(0, 32), pl.ds((s + _NW) * _SLAB, _SLAB)],
                in2.at[pl.ds((1 - p) * 32, 32)],
                in_sem,
            )

        @pl.when(i >= 2)
        def _():
            pltpu.make_async_copy(
                tt_hbm.at[pl.ds(0, 32), pl.ds(0, _SLAB)],
                in2.at[pl.ds(p * 32, 32)],
                out_sem,
            ).wait()

        transpose_slab(p)
        pltpu.async_copy(
            out2.at[pl.ds(p * (_SLAB // _PACK), _SLAB // _PACK)],
            out_hbm.at[pl.ds(s * (_SLAB // _PACK), _SLAB // _PACK)],
            out_sem,
        )
        return carry

    lax.fori_loop(0, n_w, body, 0)

    for _ in range(2):
        pltpu.make_async_copy(
            tt_hbm.at[pl.ds(0, 32), pl.ds(0, _SLAB)],
            in2.at[pl.ds(0, 32)],
            out_sem,
        ).wait()

    @pl.when(wid == _NW - 1)
    def _():
        pltpu.sync_copy(tail_hbm, out2.at[pl.ds(0, 16)])
        pltpu.sync_copy(
            out2.at[pl.ds(0, 16)],
            out_hbm.at[pl.ds(_NSLAB * _SLAB // _PACK, 16)],
        )


@functools.partial(
    pl.kernel,
    mesh=_sc_mesh,
    out_type=jax.ShapeDtypeStruct((_BATCH, _NUM_FIELDS + 1, _EMBED_DIM), jnp.float32),
    scratch_types=[
        pltpu.VMEM((_CHUNK, _NUM_FIELDS), jnp.int32),
        pltpu.VMEM((_CHUNK, _NUM_FIELDS + 1, _EMBED_DIM), jnp.float32),
        pltpu.SemaphoreType.DMA,
    ],
    compiler_params=pltpu.CompilerParams(use_tc_tiling_on_sc=False),
)
def _sc_gather(idx_hbm, num_hbm, table_hbm, out_hbm, idx_v, full_v, sem):
    wid = lax.axis_index("s") * 2 + lax.axis_index("c")

    def chunk_body(c, carry):
        base = wid * _ROWS_PER_W + c * _CHUNK
        pltpu.sync_copy(idx_hbm.at[pl.ds(base, _CHUNK)], idx_v)
        cps = [
            pltpu.async_copy(
                table_hbm.at[idx_v.at[r]],
                full_v.at[r, pl.ds(0, _NUM_FIELDS)],
                sem,
            )
            for r in range(_CHUNK)
        ]
        pltpu.sync_copy(
            num_hbm.at[pl.ds(base, _CHUNK)],
            full_v.at[pl.ds(0, _CHUNK), _NUM_FIELDS],
        )
        for cp in cps:
            cp.wait()
        pltpu.sync_copy(full_v, out_hbm.at[pl.ds(base, _CHUNK)])
        return carry

    lax.fori_loop(0, _NCHUNK, chunk_body, 0)


def kernel(categorical_x, numerical_x, table, W, b):
    offsets = jnp.asarray(_OFFSETS)[None, :]
    wt = W.T
    b2 = b[None, :]
    idx, num = pl.pallas_call(
        _tc_prep,
        out_shape=(
            jax.ShapeDtypeStruct((_BATCH, _NUM_FIELDS), jnp.int32),
            jax.ShapeDtypeStruct((_BATCH, _EMBED_DIM), jnp.float32),
        ),
    )(categorical_x, numerical_x, wt, b2, offsets)
    packed = pl.pallas_call(
        _tc_transpose,
        grid=(_NTBLK,),
        in_specs=[pl.BlockSpec((32, _TBLK), lambda i: (0, i))],
        out_specs=pl.BlockSpec((_TBLK // _PACK, 128), lambda i: (i, 0)),
        out_shape=jax.ShapeDtypeStruct((_NTBLK * _TBLK // _PACK, 128), jnp.float32),
    )(table.T)
    table_lin = packed.reshape(_NTBLK * _TBLK, _EMBED_DIM)
    out = _sc_gather(idx, num, table_lin)
    return out.reshape(_BATCH, (_NUM_FIELDS + 1) * _EMBED_DIM)


# speedup vs baseline: 1.0949x; 1.0078x over previous
import functools

import numpy as np
import jax
import jax.numpy as jnp
from jax import lax
from jax.experimental import pallas as pl
from jax.experimental.pallas import tpu as pltpu
from jax.experimental.pallas import tpu_sc as plsc

_NUM_FIELDS = 26
_EMBED_DIM = 32
_NUM_NUM = 13
_BATCH = 16384
_FIELD_SIZE = 100000
_OFFSETS = np.arange(_NUM_FIELDS, dtype=np.int32) * _FIELD_SIZE

_NW = 32
_ROWS_PER_W = _BATCH // _NW
_CHUNK = 64
_NCHUNK = _ROWS_PER_W // _CHUNK

_ROWS = 2600000
_ROWS_PAD = 2600064
_SLAB = 512
_NSLAB = (_ROWS_PAD - 128) // _SLAB
_SLAB_REM = 22
_PACK = 128 // _EMBED_DIM
_OUT_ROWS = _ROWS_PAD // _PACK


_TBLK = 65536
_NTBLK = 40
_TSUB = _TBLK // 4


def _tc_transpose(tt_ref, out_ref):
    x = tt_ref[...]
    xs = jnp.concatenate(
        [x[:, q * _TSUB : (q + 1) * _TSUB] for q in range(4)], axis=0
    )
    out_ref[...] = jnp.transpose(xs)


def _tc_prep(cat_ref, x_ref, wt_ref, b_ref, off_ref, idx_ref, num_ref):
    r = cat_ref[...] + off_ref[...]
    idx_ref[...] = (
        (r & -_TBLK) + ((r & (_TSUB - 1)) << 2) + ((r & (_TBLK - 1)) >> 14)
    )
    num_ref[...] = (
        jnp.dot(x_ref[...], wt_ref[...], preferred_element_type=jnp.float32)
        + b_ref[...]
    )


_sc_mesh = plsc.VectorSubcoreMesh(core_axis_name="c", subcore_axis_name="s")


@functools.partial(
    pl.kernel,
    mesh=_sc_mesh,
    out_type=jax.ShapeDtypeStruct((_OUT_ROWS, 128), jnp.float32),
    scratch_types=[
        pltpu.VMEM((2 * 32, _SLAB), jnp.float32),
        pltpu.VMEM((2 * (_SLAB // _PACK), 128), jnp.float32),
        pltpu.SemaphoreType.DMA,
        pltpu.SemaphoreType.DMA,
    ],
    compiler_params=pltpu.CompilerParams(
        use_tc_tiling_on_sc=True,
        needs_layout_passes=False,
        disable_bounds_checks=True,
    ),
)
def _sc_relayout(tt_hbm, tail_hbm, out_hbm, in2, out2, in_sem, out_sem):
    wid = lax.axis_index("s") * 2 + lax.axis_index("c")
    n_w = jnp.where(wid < _SLAB_REM, _NSLAB // _NW + 1, _NSLAB // _NW)
    iota = lax.iota(jnp.int32, 16)
    _UJ = 8

    def transpose_slab(p):
        def tj(jb, carry):
            for jj in range(_UJ):
                j = jb * _UJ + jj
                for k in range(8):
                    idx_d = iota + (p * 32 + 16 * (k % 2))
                    idx_r = jnp.full((16,), 0, jnp.int32) + (_PACK * j + k // 2)
                    vec = plsc.load_gather(in2, [idx_d, idx_r])
                    out2[p * (_SLAB // _PACK) + j, pl.ds(k * 16, 16)] = vec
            return carry

        lax.fori_loop(0, (_SLAB // _PACK) // _UJ, tj, 0)

    pltpu.async_copy(
        tt_hbm.at[pl.ds(0, 32), pl.ds(wid * _SLAB, _SLAB)],
        in2.at[pl.ds(0, 32)],
        in_sem,
    )

    def body(i, carry):
        p = lax.rem(i, 2)
        s = wid + _NW * i
        pltpu.make_async_copy(
            tt_hbm.at[pl.ds(0, 32), pl.ds(s * _SLAB, _SLAB)],
            in2.at[pl.ds(p * 32, 32)],
            in_sem,
        ).wait()

        @pl.when(i + 1 < n_w)
        def _():
            pltpu.async_copy(
                tt_hbm.at[pl.ds(0, 32), pl.ds((s + _NW) * _SLAB, _SLAB)],
                in2.at[pl.ds((1 - p) * 32, 32)],
                in_sem,
            )

        @pl.when(i >= 2)
        def _():
            pltpu.make_async_copy(
                tt_hbm.at[pl.ds(0, 32), pl.ds(0, _SLAB)],
                in2.at[pl.ds(p * 32, 32)],
                out_sem,
            ).wait()

        transpose_slab(p)
        pltpu.async_copy(
            out2.at[pl.ds(p * (_SLAB // _PACK), _SLAB // _PACK)],
            out_hbm.at[pl.ds(s * (_SLAB // _PACK), _SLAB // _PACK)],
            out_sem,
        )
        return carry

    lax.fori_loop(0, n_w, body, 0)

    for _ in range(2):
        pltpu.make_async_copy(
            tt_hbm.at[pl.ds(0, 32), pl.ds(0, _SLAB)],
            in2.at[pl.ds(0, 32)],
            out_sem,
        ).wait()

    @pl.when(wid == _NW - 1)
    def _():
        pltpu.sync_copy(tail_hbm, out2.at[pl.ds(0, 16)])
        pltpu.sync_copy(
            out2.at[pl.ds(0, 16)],
            out_hbm.at[pl.ds(_NSLAB * _SLAB // _PACK, 16)],
        )


@functools.partial(
    pl.kernel,
    mesh=_sc_mesh,
    out_type=jax.ShapeDtypeStruct((_BATCH, _NUM_FIELDS + 1, _EMBED_DIM), jnp.float32),
    scratch_types=[
        pltpu.VMEM((_CHUNK, _NUM_FIELDS), jnp.int32),
        pltpu.VMEM((_CHUNK, _NUM_FIELDS + 1, _EMBED_DIM), jnp.float32),
        pltpu.SemaphoreType.DMA,
    ],
    compiler_params=pltpu.CompilerParams(use_tc_tiling_on_sc=False),
)
def _sc_gather(idx_hbm, num_hbm, table_hbm, out_hbm, idx_v, full_v, sem):
    wid = lax.axis_index("s") * 2 + lax.axis_index("c")

    def chunk_body(c, carry):
        base = wid * _ROWS_PER_W + c * _CHUNK
        pltpu.sync_copy(idx_hbm.at[pl.ds(base, _CHUNK)], idx_v)
        cps = [
            pltpu.async_copy(
                table_hbm.at[idx_v.at[r]],
                full_v.at[r, pl.ds(0, _NUM_FIELDS)],
                sem,
            )
            for r in range(_CHUNK)
        ]
        pltpu.sync_copy(
            num_hbm.at[pl.ds(base, _CHUNK)],
            full_v.at[pl.ds(0, _CHUNK), _NUM_FIELDS],
        )
        for cp in cps:
            cp.wait()
        pltpu.sync_copy(full_v, out_hbm.at[pl.ds(base, _CHUNK)])
        return carry

    lax.fori_loop(0, _NCHUNK, chunk_body, 0)


def kernel(categorical_x, numerical_x, table, W, b):
    offsets = jnp.asarray(_OFFSETS)[None, :]
    wt = W.T
    b2 = b[None, :]
    idx, num = pl.pallas_call(
        _tc_prep,
        out_shape=(
            jax.ShapeDtypeStruct((_BATCH, _NUM_FIELDS), jnp.int32),
            jax.ShapeDtypeStruct((_BATCH, _EMBED_DIM), jnp.float32),
        ),
    )(categorical_x, numerical_x, wt, b2, offsets)
    packed = pl.pallas_call(
        _tc_transpose,
        grid=(_NTBLK,),
        in_specs=[pl.BlockSpec((32, _TBLK), lambda i: (0, i))],
        out_specs=pl.BlockSpec((_TBLK // _PACK, 128), lambda i: (i, 0)),
        out_shape=jax.ShapeDtypeStruct((_NTBLK * _TBLK // _PACK, 128), jnp.float32),
    )(table.T)
    table_lin = packed.reshape(_NTBLK * _TBLK, _EMBED_DIM)
    out = _sc_gather(idx, num, table_lin)
    return out.reshape(_BATCH, (_NUM_FIELDS + 1) * _EMBED_DIM)
